# Initial kernel scaffold; baseline (speedup 1.0000x reference)
import jax
import jax.numpy as jnp
from jax.experimental import pallas as pl


def kernel(features, cam_idx, u_idx, v_idx):
    raise NotImplementedError("write your pallas kernel here")


# speedup vs baseline: 3.4129x; 3.4129x over previous
import functools

import jax
import jax.numpy as jnp
from jax import lax
from jax.experimental import pallas as pl
from jax.experimental.pallas import tpu as pltpu
from jax.experimental.pallas import tpu_sc as plsc

B, N, C, H, W = 4, 6, 64, 64, 176
NH = N * H
NX, NY, NZ = 200, 200, 4
V = NX * NY * NZ
VROW = NY * NZ
VPAD = 1024
VP = NX * VPAD

NC, NS = 2, 16
NW = NC * NS
PAIRS = B * C
PPT = PAIRS // NW

NXB = 13
CHUNK = 2560
NCHUNK = VP // CHUNK


def _fr_kernel(feat_hbm, cam_hbm, u_hbm, v_hbm, out_hbm,
               sidx, table, idxb0, idxb1, outb0, outb1,
               camb, ub, vb, camb2, ub2, vb2, packedb, physb,
               sem_t, sem_i0, sem_i1, sem_o0, sem_o1):
    idxb = (idxb0, idxb1)
    outb = (outb0, outb1)
    sem_i = (sem_i0, sem_i1)
    sem_o = (sem_o0, sem_o1)
    cid = lax.axis_index("c")
    sid = lax.axis_index("s")
    wid = sid * NC + cid

    stage = ((camb, ub, vb), (camb2, ub2, vb2))
    p1sem = (sem_i0, sem_i1)

    def p1_issue(k, j):
        voff = (sid + k * NS) * VROW
        cb, ub_, vb_ = stage[j]
        pltpu.async_copy(cam_hbm.at[pl.ds(voff, VROW)], cb, p1sem[j])
        pltpu.async_copy(v_hbm.at[pl.ds(voff, VROW)], vb_, p1sem[j])
        pltpu.async_copy(u_hbm.at[pl.ds(voff, VROW)], ub_, p1sem[j])

    def p1_compute(k, j):
        nx = sid + k * NS
        cb, ub_, vb_ = stage[j]
        voff = nx * VROW
        pltpu.make_async_copy(cam_hbm.at[pl.ds(voff, VROW)], cb,
                              p1sem[j]).wait()
        pltpu.make_async_copy(v_hbm.at[pl.ds(voff, VROW)], vb_,
                              p1sem[j]).wait()
        pltpu.make_async_copy(u_hbm.at[pl.ds(voff, VROW)], ub_,
                              p1sem[j]).wait()

        @plsc.parallel_loop(0, VROW, 16, unroll=8)
        def _(i):
            packedb[pl.ds(i, 16)] = (cb[pl.ds(i, 16)] * (H * 256)
                                     + vb_[pl.ds(i, 16)] * 256
                                     + ub_[pl.ds(i, 16)])

        @plsc.parallel_loop(0, VPAD, 16, unroll=8)
        def _(i):
            iv = lax.iota(jnp.int32, 16) + i
            tc = lax.shift_right_logical(iv, 9)
            nz = lax.bitwise_and(lax.shift_right_logical(iv, 7), 3)
            ny = lax.shift_left(tc, 7) + lax.bitwise_and(iv, 127)
            vloc = lax.shift_left(ny, 2) + nz
            pv = jnp.where(ny < NY, vloc, 0)
            physb[pl.ds(i, 16)] = plsc.load_gather(packedb, [pv])

        pltpu.sync_copy(physb, sidx.at[pl.ds(nx * VPAD, VPAD)])

    def p1_pair(q, carry):
        for j in range(2):
            k = q * 2 + j

            @pl.when(sid + k * NS < NX)
            def _():
                @pl.when(sid + (k + 1) * NS < NX)
                def _():
                    p1_issue(k + 1, 1 - j)

                p1_compute(k, j)
        return carry

    def load_table(pg):
        b = pg // C
        ch = pg % C
        return [
            pltpu.async_copy(feat_hbm.at[b, n, ch],
                             table.at[pl.ds(n * H, H), :], sem_t)
            for n in range(N)
        ]

    load_table(wid * PPT)

    @pl.when(sid < NX)
    def _():
        p1_issue(0, 0)

    lax.fori_loop(0, (NXB + 1) // 2, p1_pair, 0)
    plsc.subcore_barrier()

    def do_pair(p, carry):
        pg = wid * PPT + p
        out_base = pg * VP

        @pl.when(p > 0)
        def _():
            load_table(pg)

        for n in range(N):
            pltpu.make_async_copy(feat_hbm.at[0, n, 0],
                                  table.at[pl.ds(n * H, H), :], sem_t).wait()

        pltpu.async_copy(sidx.at[pl.ds(0, CHUNK)], idxb[0], sem_i[0])
        pltpu.async_copy(sidx.at[pl.ds(CHUNK, CHUNK)], idxb[1], sem_i[1])

        def chunk2(t2, carry):
            for j in range(2):
                t = t2 + j
                pltpu.make_async_copy(sidx.at[pl.ds(t * CHUNK, CHUNK)],
                                      idxb[j], sem_i[j]).wait()

                @pl.when(t >= 2)
                def _():
                    pltpu.make_async_copy(
                        outb[j],
                        out_hbm.at[pl.ds(out_base + (t - 2) * CHUNK, CHUNK)],
                        sem_o[j]).wait()

                @plsc.parallel_loop(0, CHUNK, 16, unroll=8)
                def _(i):
                    packed = idxb[j][pl.ds(i, 16)]
                    row = lax.shift_right_logical(packed, 8)
                    col = lax.bitwise_and(packed, 255)
                    outb[j][pl.ds(i, 16)] = plsc.load_gather(table, [row, col])

                pltpu.async_copy(outb[j],
                                 out_hbm.at[pl.ds(out_base + t * CHUNK, CHUNK)],
                                 sem_o[j])

                @pl.when(t + 2 < NCHUNK)
                def _():
                    pltpu.async_copy(sidx.at[pl.ds((t + 2) * CHUNK, CHUNK)],
                                     idxb[j], sem_i[j])
            return carry

        lax.fori_loop(0, NCHUNK // 2, lambda q, c2: chunk2(q * 2, c2), 0)

        for j in range(2):
            t = NCHUNK - 2 + j
            pltpu.make_async_copy(outb[j],
                                  out_hbm.at[pl.ds(out_base + t * CHUNK, CHUNK)],
                                  sem_o[j]).wait()
        return carry

    lax.fori_loop(0, PPT, do_pair, 0)


def kernel(features, cam_idx, u_idx, v_idx):
    mesh = plsc.VectorSubcoreMesh(core_axis_name="c", subcore_axis_name="s",
                                  num_cores=NC, num_subcores=NS)

    fr = functools.partial(
        pl.kernel,
        out_type=jax.ShapeDtypeStruct((B * C * VP,), jnp.float32),
        mesh=mesh,
        compiler_params=pltpu.CompilerParams(needs_layout_passes=False),
        scratch_types=[
            pltpu.VMEM_SHARED((VP,), jnp.int32),
            pltpu.VMEM((NH, W), jnp.float32),
            pltpu.VMEM((CHUNK,), jnp.int32),
            pltpu.VMEM((CHUNK,), jnp.int32),
            pltpu.VMEM((CHUNK,), jnp.float32),
            pltpu.VMEM((CHUNK,), jnp.float32),
            pltpu.VMEM((VROW,), jnp.int32),
            pltpu.VMEM((VROW,), jnp.int32),
            pltpu.VMEM((VROW,), jnp.int32),
            pltpu.VMEM((VROW,), jnp.int32),
            pltpu.VMEM((VROW,), jnp.int32),
            pltpu.VMEM((VROW,), jnp.int32),
            pltpu.VMEM((VROW,), jnp.int32),
            pltpu.VMEM((VPAD,), jnp.int32),
            pltpu.SemaphoreType.DMA,
            pltpu.SemaphoreType.DMA,
            pltpu.SemaphoreType.DMA,
            pltpu.SemaphoreType.DMA,
            pltpu.SemaphoreType.DMA,
        ],
    )(_fr_kernel)

    out = fr(features, cam_idx, u_idx, v_idx)
    x = out.reshape(B, C, NX, 2, NZ, 128)
    x = jnp.transpose(x, (0, 1, 2, 3, 5, 4))
    x = x.reshape(B, C, NX, 256, NZ)
    return x[:, :, :, :NY, :]
